# Initial kernel scaffold; baseline (speedup 1.0000x reference)
import jax
import jax.numpy as jnp
from jax.experimental import pallas as pl


def kernel(feat, head_W, head_a, head_gamma, head_beta, out_W, out_a, out_gamma, out_beta, res_W, res_b, edge):
    raise NotImplementedError("write your pallas kernel here")


# speedup vs baseline: 1.1470x; 1.1470x over previous
import functools

import jax
import jax.numpy as jnp
from jax.experimental import pallas as pl
from jax.experimental.pallas import tpu as pltpu

ALPHA = 0.2
BN_EPS = 1e-5
_PREC = jax.lax.Precision.HIGHEST


def _row_block(n):
    for nb in (1000, 500, 250, 125):
        if n % nb == 0 and nb % 8 == 0:
            return nb
    return n


def _proj_body(feat_ref, w_ref, asrc_ref, adst_ref, msg_ref, as_ref, ad_ref):
    msg = jnp.dot(feat_ref[...], w_ref[...], preferred_element_type=jnp.float32,
                  precision=_PREC)
    msg_ref[...] = msg
    as_ref[...] = jnp.dot(msg, asrc_ref[...], preferred_element_type=jnp.float32,
                          precision=_PREC)
    ad_ref[...] = jnp.dot(msg, adst_ref[...], preferred_element_type=jnp.float32,
                          precision=_PREC)


def _proj(feat, w_all, a_src, a_dst):
    n, din = feat.shape
    dall = w_all.shape[1]
    h = a_src.shape[1]
    nb = _row_block(n)
    grid = n // nb
    return pl.pallas_call(
        _proj_body,
        grid=(grid,),
        in_specs=[
            pl.BlockSpec((nb, din), lambda i: (i, 0)),
            pl.BlockSpec((din, dall), lambda i: (0, 0)),
            pl.BlockSpec((dall, h), lambda i: (0, 0)),
            pl.BlockSpec((dall, h), lambda i: (0, 0)),
        ],
        out_specs=[
            pl.BlockSpec((nb, dall), lambda i: (i, 0)),
            pl.BlockSpec((nb, h), lambda i: (i, 0)),
            pl.BlockSpec((nb, h), lambda i: (i, 0)),
        ],
        out_shape=[
            jax.ShapeDtypeStruct((n, dall), jnp.float32),
            jax.ShapeDtypeStruct((n, h), jnp.float32),
            jax.ShapeDtypeStruct((n, h), jnp.float32),
        ],
    )(feat, w_all, a_src, a_dst)


def _stats_body(x_ref, sum_ref, sq_ref):
    i = pl.program_id(0)
    x = x_ref[...]
    s = jnp.sum(x, axis=0, keepdims=True)
    q = jnp.sum(x * x, axis=0, keepdims=True)

    @pl.when(i == 0)
    def _():
        sum_ref[...] = s
        sq_ref[...] = q

    @pl.when(i != 0)
    def _():
        sum_ref[...] += s
        sq_ref[...] += q


def _stats(x):
    n, c = x.shape
    nb = _row_block(n)
    return pl.pallas_call(
        _stats_body,
        grid=(n // nb,),
        in_specs=[pl.BlockSpec((nb, c), lambda i: (i, 0))],
        out_specs=[pl.BlockSpec((1, c), lambda i: (0, 0)),
                   pl.BlockSpec((1, c), lambda i: (0, 0))],
        out_shape=[jax.ShapeDtypeStruct((1, c), jnp.float32),
                   jax.ShapeDtypeStruct((1, c), jnp.float32)],
    )(x)


def _l2_body(agg_ref, s_ref, c_ref, w_ref, a_ref, msg_ref, as_ref, ad_ref):
    xhat = agg_ref[...] * s_ref[...] + c_ref[...]
    msg = jnp.dot(xhat, w_ref[...], preferred_element_type=jnp.float32,
                  precision=_PREC)
    msg_ref[...] = msg
    dout = msg.shape[1]
    as_ref[...] = jnp.dot(msg, a_ref[:dout, :], preferred_element_type=jnp.float32,
                          precision=_PREC)
    ad_ref[...] = jnp.dot(msg, a_ref[dout:, :], preferred_element_type=jnp.float32,
                          precision=_PREC)


def _l2(agg, s, c, w, a):
    n, dall = agg.shape
    dout = w.shape[1]
    nb = _row_block(n)
    return pl.pallas_call(
        _l2_body,
        grid=(n // nb,),
        in_specs=[
            pl.BlockSpec((nb, dall), lambda i: (i, 0)),
            pl.BlockSpec((1, dall), lambda i: (0, 0)),
            pl.BlockSpec((1, dall), lambda i: (0, 0)),
            pl.BlockSpec((dall, dout), lambda i: (0, 0)),
            pl.BlockSpec((2 * dout, 1), lambda i: (0, 0)),
        ],
        out_specs=[
            pl.BlockSpec((nb, dout), lambda i: (i, 0)),
            pl.BlockSpec((nb, 1), lambda i: (i, 0)),
            pl.BlockSpec((nb, 1), lambda i: (i, 0)),
        ],
        out_shape=[
            jax.ShapeDtypeStruct((n, dout), jnp.float32),
            jax.ShapeDtypeStruct((n, 1), jnp.float32),
            jax.ShapeDtypeStruct((n, 1), jnp.float32),
        ],
    )(agg, s, c, w, a)


def _final_body(agg_ref, feat_ref, rwt_ref, s_ref, c_ref, rb_ref, out_ref):
    res = jnp.dot(feat_ref[...], rwt_ref[...], preferred_element_type=jnp.float32,
                  precision=_PREC)
    out_ref[...] = agg_ref[...] * s_ref[...] + c_ref[...] + res + rb_ref[...]


def _final(agg2, feat, rwt, s, c, rb):
    n, dout = agg2.shape
    din = feat.shape[1]
    nb = _row_block(n)
    return pl.pallas_call(
        _final_body,
        grid=(n // nb,),
        in_specs=[
            pl.BlockSpec((nb, dout), lambda i: (i, 0)),
            pl.BlockSpec((nb, din), lambda i: (i, 0)),
            pl.BlockSpec((din, dout), lambda i: (0, 0)),
            pl.BlockSpec((1, dout), lambda i: (0, 0)),
            pl.BlockSpec((1, dout), lambda i: (0, 0)),
            pl.BlockSpec((1, dout), lambda i: (0, 0)),
        ],
        out_specs=pl.BlockSpec((nb, dout), lambda i: (i, 0)),
        out_shape=jax.ShapeDtypeStruct((n, dout), jnp.float32),
    )(agg2, feat, rwt, s, c, rb)


def _edges(asrc, adst, msg, src, dst):
    n = msg.shape[0]
    h = asrc.shape[1]
    d = msg.shape[1] // h
    t = asrc[src] + adst[dst]
    scores = jnp.where(t >= 0, t, ALPHA * t)
    m = scores.max(axis=0)
    expa = jnp.exp(scores - m)
    exp_sum = jnp.zeros((n, h), jnp.float32).at[dst].add(expa) + 1e-10
    coeff = expa / exp_sum[dst]
    msg3 = msg.reshape(n, h, d)
    agg = jnp.zeros((n, h, d), jnp.float32).at[dst].add(
        msg3[src] * coeff[:, :, None])
    return agg.reshape(n, h * d)


def kernel(feat, head_W, head_a, head_gamma, head_beta, out_W, out_a,
           out_gamma, out_beta, res_W, res_b, edge):
    n, din = feat.shape
    h, _, dh = head_W.shape
    dout = out_W.shape[1]
    src, dst = edge[0], edge[1]

    w_all = jnp.transpose(head_W, (1, 0, 2)).reshape(din, h * dh)
    eye = jnp.eye(h, dtype=jnp.float32)
    a_src = (head_a[:, :dh, 0][:, :, None] * eye[:, None, :]).reshape(h * dh, h)
    a_dst = (head_a[:, dh:, 0][:, :, None] * eye[:, None, :]).reshape(h * dh, h)

    msg1, as1, ad1 = _proj(feat, w_all, a_src, a_dst)
    agg1 = _edges(as1, ad1, msg1, src, dst)

    sum1, sq1 = _stats(agg1)
    mean1 = sum1 / n
    var1 = sq1 / n - mean1 * mean1
    g1 = head_gamma.reshape(1, h * dh)
    b1 = head_beta.reshape(1, h * dh)
    s1 = g1 / jnp.sqrt(var1 + BN_EPS)
    c1 = b1 - mean1 * s1

    msg2, as2, ad2 = _l2(agg1, s1, c1, out_W, out_a)
    agg2 = _edges(as2, ad2, msg2, src, dst)

    sum2, sq2 = _stats(agg2)
    mean2 = sum2 / n
    var2 = sq2 / n - mean2 * mean2
    s2 = out_gamma.reshape(1, dout) / jnp.sqrt(var2 + BN_EPS)
    c2 = out_beta.reshape(1, dout) - mean2 * s2

    return _final(agg2, feat, res_W.T, s2, c2, res_b.reshape(1, dout))
